# baseline (speedup 1.0000x reference)
import functools
import math

import jax
import jax.numpy as jnp
import numpy as np
from jax import lax
from jax.experimental import pallas as pl
from jax.experimental.pallas import tpu as pltpu
from jax.experimental.pallas import tpu_sc as plsc

_SPARSITY = 0.05
_INT_MIN = np.int32(-(2**31))
_GROUP = 16
_NBKT = 256


def _sc_body(in_hbm, out_hbm, buf, hist, *, rows_per_worker, e, k):
    nc = 2
    wid = lax.axis_index("s") * nc + lax.axis_index("c")
    iota = lax.iota(jnp.int32, 16)
    ones = jnp.ones((16,), jnp.int32)
    zeros16 = jnp.zeros((16,), jnp.int32)
    lane_hist_base = iota * _NBKT
    n_groups = rows_per_worker // _GROUP
    ev = e // 8

    lane_row_base = iota * e

    def group_body(t, _):
        grow = wid * rows_per_worker + t * _GROUP
        pltpu.sync_copy(in_hbm.at[pl.ds(grow * e, _GROUP * e)], buf)

        prefix = zeros16
        kp = jnp.full((16,), k, jnp.int32)

        for p in range(4):
            shift = 24 - 8 * p

            def clear_body(i, _):
                for j in range(8):
                    hist[pl.ds(i * 128 + j * 16, 16)] = zeros16
                return _

            lax.fori_loop(0, (16 * _NBKT) // 128, clear_body, None)

            def hist_body(i, _, prefix=prefix, shift=shift, p=p):
                for j in range(8):
                    col = lane_row_base + (i * 8 + j)
                    v = plsc.load_gather(buf, [col])
                    s = plsc.bitcast(v, jnp.int32)
                    keyu = s ^ ((s >> 31) | _INT_MIN)
                    digit = lax.shift_right_logical(keyu, shift) & np.int32(0xFF)
                    sidx = lane_hist_base + digit
                    if p == 0:
                        plsc.addupdate_scatter(hist, [sidx], ones)
                    else:
                        pref_here = lax.shift_right_logical(keyu, shift + 8)
                        m = pref_here == prefix
                        plsc.addupdate_scatter(hist, [sidx], ones, mask=m)
                return _

            lax.fori_loop(0, ev, hist_body, None)

            def scan_body(r, bk, kp=kp):
                b_vec, kp_vec = bk
                lane_r = iota == r
                kpr = jnp.sum(jnp.where(lane_r, kp, 0))
                carry = jnp.int32(0)
                found = jnp.bool_(False)
                vstar = jnp.int32(0)
                carry_at = jnp.int32(0)
                for v in range(15, -1, -1):
                    h_v = hist[pl.ds(r * _NBKT + v * 16, 16)]
                    sv = jnp.sum(h_v)
                    new = carry + sv
                    crossed = jnp.logical_and(jnp.logical_not(found), new >= kpr)
                    vstar = jnp.where(crossed, jnp.int32(v), vstar)
                    carry_at = jnp.where(crossed, carry, carry_at)
                    found = jnp.logical_or(found, crossed)
                    carry = new
                hstar = hist[pl.ds(r * _NBKT + vstar * 16, 16)]
                rev = lax.rev(hstar, (0,))
                pre = plsc.cumsum(rev)
                thr_s = kpr - carry_at
                condm = pre >= thr_s
                nfalse = jnp.sum(jnp.where(condm, 0, 1))
                s_above = carry_at + jnp.max(jnp.where(condm, 0, pre))
                bdig = vstar * 16 + (jnp.int32(15) - nfalse)
                b_vec = jnp.where(lane_r, bdig, b_vec)
                kp_vec = jnp.where(lane_r, kpr - s_above, kp_vec)
                return b_vec, kp_vec

            b_vec, kp = lax.fori_loop(0, 16, scan_body, (zeros16, kp))
            prefix = (prefix << 8) | b_vec

        thr_i = prefix ^ _INT_MIN

        def mask_body(i, _, thr_i=thr_i):
            for j in range(8):
                col = lane_row_base + (i * 8 + j)
                v = plsc.load_gather(buf, [col])
                s = plsc.bitcast(v, jnp.int32)
                ki = s ^ ((s >> 31) & np.int32(0x7FFFFFFF))
                outv = jnp.where(ki >= thr_i, v, jnp.float32(0.0))
                plsc.store_scatter(buf, [col], outv)
            return _

        lax.fori_loop(0, ev, mask_body, None)
        pltpu.sync_copy(buf, out_hbm.at[pl.ds(grow * e, _GROUP * e)])
        return _

    lax.fori_loop(0, n_groups, group_body, None)


def kernel(tensor):
    b, f, e = tensor.shape
    k = int(math.ceil(_SPARSITY * e))
    rows = b * f
    t = tensor.reshape(rows * e)
    n_workers = 32
    rows_per_worker = rows // n_workers
    mesh = plsc.VectorSubcoreMesh(core_axis_name="c", subcore_axis_name="s")
    body = functools.partial(
        _sc_body, rows_per_worker=rows_per_worker, e=e, k=k
    )
    out = pl.kernel(
        body,
        out_type=jax.ShapeDtypeStruct((rows * e,), jnp.float32),
        mesh=mesh,
        compiler_params=pltpu.CompilerParams(
            use_tc_tiling_on_sc=False, needs_layout_passes=False
        ),
        scratch_types=[
            pltpu.VMEM((_GROUP * e,), jnp.float32),
            pltpu.VMEM((16 * _NBKT,), jnp.int32),
        ],
    )(t)
    return out.reshape(b, f, e)


# speedup vs baseline: 1.9979x; 1.9979x over previous
import functools
import math

import jax
import jax.numpy as jnp
import numpy as np
from jax import lax
from jax.experimental import pallas as pl
from jax.experimental.pallas import tpu as pltpu
from jax.experimental.pallas import tpu_sc as plsc

_SPARSITY = 0.05
_INT_MIN = np.int32(-(2**31))
_GROUP = 16
_NBKT = 256


def _sc_body(in_hbm, out_hbm, buf, hist, *, rows_per_worker, e, k):
    nc = 2
    wid = lax.axis_index("s") * nc + lax.axis_index("c")
    iota = lax.iota(jnp.int32, 16)
    ones = jnp.ones((16,), jnp.int32)
    zeros16 = jnp.zeros((16,), jnp.int32)
    lane_hist_base = iota * _NBKT
    n_groups = rows_per_worker // _GROUP
    ev = e // 8

    assert e & (e - 1) == 0
    row_base = iota * e

    def diag_idx(e0):
        return row_base + ((iota + e0) & (e - 1))

    def group_body(t, _):
        grow = wid * rows_per_worker + t * _GROUP
        pltpu.sync_copy(in_hbm.at[pl.ds(grow * e, _GROUP * e)], buf)

        prefix = zeros16
        kp = jnp.full((16,), k, jnp.int32)

        for p in range(4):
            shift = 24 - 8 * p

            def clear_body(i, _):
                for j in range(8):
                    hist[pl.ds(i * 128 + j * 16, 16)] = zeros16
                return _

            lax.fori_loop(0, (16 * _NBKT) // 128, clear_body, None)

            def hist_body(i, _, prefix=prefix, shift=shift, p=p):
                for j in range(8):
                    idx = diag_idx(i * 8 + j)
                    v = plsc.load_gather(buf, [idx])
                    s = plsc.bitcast(v, jnp.int32)
                    keyu = s ^ ((s >> 31) | _INT_MIN)
                    digit = lax.shift_right_logical(keyu, shift) & np.int32(0xFF)
                    sidx = (digit << 4) | iota
                    if p == 0:
                        plsc.addupdate_scatter(hist, [sidx], ones)
                    else:
                        pref_here = lax.shift_right_logical(keyu, shift + 8)
                        m = pref_here == prefix
                        plsc.addupdate_scatter(hist, [sidx], ones, mask=m)
                return _

            lax.fori_loop(0, ev, hist_body, None)

            def scan_body(i, st, kp=kp):
                carry, found, bdig, s_above = st
                for j in range(8):
                    d = 255 - (i * 8 + j)
                    h_d = hist[pl.ds(d * 16, 16)]
                    new = carry + h_d
                    crossed = jnp.logical_and(jnp.logical_not(found), new >= kp)
                    bdig = jnp.where(crossed, jnp.int32(d), bdig)
                    s_above = jnp.where(crossed, carry, s_above)
                    found = jnp.logical_or(found, crossed)
                    carry = new
                return carry, found, bdig, s_above

            _c, _f, b_vec, s_above = lax.fori_loop(
                0, _NBKT // 8, scan_body,
                (zeros16, jnp.zeros((16,), jnp.bool_), zeros16, zeros16),
            )
            prefix = (prefix << 8) | b_vec
            kp = kp - s_above

        thr_i = prefix ^ _INT_MIN

        def mask_body(i, _, thr_i=thr_i):
            for j in range(8):
                idx = diag_idx(i * 8 + j)
                v = plsc.load_gather(buf, [idx])
                s = plsc.bitcast(v, jnp.int32)
                ki = s ^ ((s >> 31) & np.int32(0x7FFFFFFF))
                outv = jnp.where(ki >= thr_i, v, jnp.float32(0.0))
                plsc.store_scatter(buf, [idx], outv)
            return _

        lax.fori_loop(0, ev, mask_body, None)
        pltpu.sync_copy(buf, out_hbm.at[pl.ds(grow * e, _GROUP * e)])
        return _

    lax.fori_loop(0, n_groups, group_body, None)


def kernel(tensor):
    b, f, e = tensor.shape
    k = int(math.ceil(_SPARSITY * e))
    rows = b * f
    t = tensor.reshape(rows * e)
    n_workers = 32
    rows_per_worker = rows // n_workers
    mesh = plsc.VectorSubcoreMesh(core_axis_name="c", subcore_axis_name="s")
    body = functools.partial(
        _sc_body, rows_per_worker=rows_per_worker, e=e, k=k
    )
    out = pl.kernel(
        body,
        out_type=jax.ShapeDtypeStruct((rows * e,), jnp.float32),
        mesh=mesh,
        compiler_params=pltpu.CompilerParams(
            use_tc_tiling_on_sc=False, needs_layout_passes=False
        ),
        scratch_types=[
            pltpu.VMEM((_GROUP * e,), jnp.float32),
            pltpu.VMEM((16 * _NBKT,), jnp.int32),
        ],
    )(t)
    return out.reshape(b, f, e)


# speedup vs baseline: 6.7587x; 3.3829x over previous
import functools
import math

import jax
import jax.numpy as jnp
import numpy as np
from jax import lax
from jax.experimental import pallas as pl
from jax.experimental.pallas import tpu as pltpu
from jax.experimental.pallas import tpu_sc as plsc

_SPARSITY = 0.05
_INT_MIN = np.int32(-(2**31))
_GROUP = 16
_NBKT = 256


def _sc_body(in_hbm, out_hbm, buf, hist, *, rows_per_worker, e, k):
    nc = 2
    wid = lax.axis_index("s") * nc + lax.axis_index("c")
    iota = lax.iota(jnp.int32, 16)
    ones = jnp.ones((16,), jnp.int32)
    zeros16 = jnp.zeros((16,), jnp.int32)
    lane_hist_base = iota * _NBKT
    n_groups = rows_per_worker // _GROUP
    ev = e // 8

    assert e & (e - 1) == 0
    row_base = iota * e

    def diag_idx(e0):
        return row_base + ((iota + e0) & (e - 1))

    def group_body(t, _):
        grow = wid * rows_per_worker + t * _GROUP
        pltpu.sync_copy(in_hbm.at[pl.ds(grow * e, _GROUP * e)], buf)

        prefix = zeros16
        kp = jnp.full((16,), k, jnp.int32)

        for p in range(4):
            shift = 24 - 8 * p

            @plsc.parallel_loop(0, 16 * _NBKT, 16, unroll=8)
            def _clear(i):
                hist[pl.ds(i, 16)] = zeros16

            def hist_body(i, prefix=prefix, shift=shift, p=p):
                idx = diag_idx(i)
                v = plsc.load_gather(buf, [idx])
                s = plsc.bitcast(v, jnp.int32)
                keyu = s ^ ((s >> 31) | _INT_MIN)
                digit = lax.shift_right_logical(keyu, shift) & np.int32(0xFF)
                sidx = (digit << 4) | iota
                if p == 0:
                    plsc.addupdate_scatter(hist, [sidx], ones)
                else:
                    pref_here = lax.shift_right_logical(keyu, shift + 8)
                    m = pref_here == prefix
                    plsc.addupdate_scatter(hist, [sidx], ones, mask=m)

            plsc.parallel_loop(0, e, 1, unroll=16)(hist_body)

            def scan_body(i, st, kp=kp):
                carry, found, bdig, s_above = st
                d = 255 - i
                h_d = hist[pl.ds(d * 16, 16)]
                new = carry + h_d
                crossed = jnp.logical_and(jnp.logical_not(found), new >= kp)
                bdig = jnp.where(crossed, jnp.int32(d), bdig)
                s_above = jnp.where(crossed, carry, s_above)
                found = jnp.logical_or(found, crossed)
                carry = new
                return carry, found, bdig, s_above

            _c, _f, b_vec, s_above = plsc.parallel_loop(
                0, _NBKT, 1, unroll=8,
                carry=(zeros16, jnp.zeros((16,), jnp.bool_), zeros16, zeros16),
            )(scan_body)
            prefix = (prefix << 8) | b_vec
            kp = kp - s_above

        thr_i = prefix ^ _INT_MIN

        def mask_body(i, thr_i=thr_i):
            idx = diag_idx(i)
            v = plsc.load_gather(buf, [idx])
            s = plsc.bitcast(v, jnp.int32)
            ki = s ^ ((s >> 31) & np.int32(0x7FFFFFFF))
            outv = jnp.where(ki >= thr_i, v, jnp.float32(0.0))
            plsc.store_scatter(buf, [idx], outv)

        plsc.parallel_loop(0, e, 1, unroll=16)(mask_body)
        pltpu.sync_copy(buf, out_hbm.at[pl.ds(grow * e, _GROUP * e)])
        return _

    lax.fori_loop(0, n_groups, group_body, None)


def kernel(tensor):
    b, f, e = tensor.shape
    k = int(math.ceil(_SPARSITY * e))
    rows = b * f
    t = tensor.reshape(rows * e)
    n_workers = 32
    rows_per_worker = rows // n_workers
    mesh = plsc.VectorSubcoreMesh(core_axis_name="c", subcore_axis_name="s")
    body = functools.partial(
        _sc_body, rows_per_worker=rows_per_worker, e=e, k=k
    )
    out = pl.kernel(
        body,
        out_type=jax.ShapeDtypeStruct((rows * e,), jnp.float32),
        mesh=mesh,
        compiler_params=pltpu.CompilerParams(
            use_tc_tiling_on_sc=False, needs_layout_passes=False
        ),
        scratch_types=[
            pltpu.VMEM((_GROUP * e,), jnp.float32),
            pltpu.VMEM((16 * _NBKT,), jnp.int32),
        ],
    )(t)
    return out.reshape(b, f, e)


# speedup vs baseline: 10.4230x; 1.5422x over previous
import functools
import math

import jax
import jax.numpy as jnp
import numpy as np
from jax import lax
from jax.experimental import pallas as pl
from jax.experimental.pallas import tpu as pltpu
from jax.experimental.pallas import tpu_sc as plsc

_SPARSITY = 0.05
_INT_MIN = np.int32(-(2**31))
_GROUP = 16
_NBKT = 256


def _sc_body(in_hbm, out_hbm, buf, hist, *, rows_per_worker, e, k):
    nc = 2
    wid = lax.axis_index("s") * nc + lax.axis_index("c")
    iota = lax.iota(jnp.int32, 16)
    ones = jnp.ones((16,), jnp.int32)
    zeros16 = jnp.zeros((16,), jnp.int32)
    lane_hist_base = iota * _NBKT
    n_groups = rows_per_worker // _GROUP
    ev = e // 8

    assert e & (e - 1) == 0
    row_base = iota * e

    def diag_idx(e0):
        return row_base + ((iota + e0) & (e - 1))

    def group_body(t, _):
        grow = wid * rows_per_worker + t * _GROUP
        pltpu.sync_copy(in_hbm.at[pl.ds(grow * e, _GROUP * e)], buf)

        prefix = zeros16
        kp = jnp.full((16,), k, jnp.int32)

        for p in range(4):
            shift = 24 - 8 * p

            @plsc.parallel_loop(0, 16 * _NBKT, 16, unroll=8)
            def _clear(i):
                hist[pl.ds(i, 16)] = zeros16

            def hist_body(i, prefix=prefix, shift=shift, p=p):
                idx = diag_idx(i)
                v = plsc.load_gather(buf, [idx])
                s = plsc.bitcast(v, jnp.int32)
                keyu = s ^ ((s >> 31) | _INT_MIN)
                digit = lax.shift_right_logical(keyu, shift) & np.int32(0xFF)
                sidx = (digit << 4) | iota
                if p == 0:
                    plsc.addupdate_scatter(hist, [sidx], ones)
                else:
                    pref_here = lax.shift_right_logical(keyu, shift + 8)
                    m = pref_here == prefix
                    plsc.addupdate_scatter(hist, [sidx], ones, mask=m)

            plsc.parallel_loop(0, e, 1, unroll=16)(hist_body)

            def scan_body(i, st, kp=kp):
                carry, found, bdig, s_above = st
                d = 255 - i
                h_d = hist[pl.ds(d * 16, 16)]
                new = carry + h_d
                crossed = jnp.logical_and(jnp.logical_not(found), new >= kp)
                bdig = jnp.where(crossed, jnp.int32(d), bdig)
                s_above = jnp.where(crossed, carry, s_above)
                found = jnp.logical_or(found, crossed)
                carry = new
                return carry, found, bdig, s_above

            _c, _f, b_vec, s_above = plsc.parallel_loop(
                0, _NBKT, 1, unroll=8,
                carry=(zeros16, jnp.zeros((16,), jnp.bool_), zeros16, zeros16),
            )(scan_body)
            prefix = (prefix << 8) | b_vec
            kp = kp - s_above

        thr_i = prefix ^ _INT_MIN

        def mask_body(i, thr_i=thr_i):
            idx = diag_idx(i)
            v = plsc.load_gather(buf, [idx])
            s = plsc.bitcast(v, jnp.int32)
            ki = s ^ ((s >> 31) & np.int32(0x7FFFFFFF))
            outv = jnp.where(ki >= thr_i, v, jnp.float32(0.0))
            plsc.store_scatter(buf, [idx], outv)

        plsc.parallel_loop(0, e, 1, unroll=16)(mask_body)
        pltpu.sync_copy(buf, out_hbm.at[pl.ds(grow * e, _GROUP * e)])
        return _

    lax.fori_loop(0, n_groups, group_body, None)


def _tc_body(x_ref, o_ref, *, k):
    x = x_ref[...]
    s = jax.lax.bitcast_convert_type(x, jnp.int32)
    ki = s ^ ((s >> 31) & np.int32(0x7FFFFFFF))
    rows = x.shape[0]
    p = jnp.zeros((rows, 1), jnp.int32)
    for bit in range(31, -1, -1):
        m = np.int32(np.uint32(1 << bit).astype(np.int32))
        pt = p | m
        thr = pt ^ _INT_MIN
        cnt = jnp.sum((ki >= thr).astype(jnp.int32), axis=1, keepdims=True)
        p = jnp.where(cnt >= k, pt, p)
    thr = p ^ _INT_MIN
    o_ref[...] = jnp.where(ki >= thr, x, 0.0)


def _run_sc(t_flat, rows, e, k):
    n_workers = 32
    mesh = plsc.VectorSubcoreMesh(core_axis_name="c", subcore_axis_name="s")
    body = functools.partial(
        _sc_body, rows_per_worker=rows // n_workers, e=e, k=k
    )
    return pl.kernel(
        body,
        out_type=jax.ShapeDtypeStruct((rows * e,), jnp.float32),
        mesh=mesh,
        compiler_params=pltpu.CompilerParams(
            use_tc_tiling_on_sc=False, needs_layout_passes=False
        ),
        scratch_types=[
            pltpu.VMEM((_GROUP * e,), jnp.float32),
            pltpu.VMEM((16 * _NBKT,), jnp.int32),
        ],
    )(t_flat)


def _run_tc(t2d, e, k):
    rows = t2d.shape[0]
    block_rows = 256
    return pl.pallas_call(
        functools.partial(_tc_body, k=k),
        grid=(rows // block_rows,),
        in_specs=[pl.BlockSpec((block_rows, e), lambda i: (i, 0))],
        out_specs=pl.BlockSpec((block_rows, e), lambda i: (i, 0)),
        out_shape=jax.ShapeDtypeStruct((rows, e), jnp.float32),
    )(t2d)


_SC_ROWS = 2560


def kernel(tensor):
    b, f, e = tensor.shape
    k = int(math.ceil(_SPARSITY * e))
    rows = b * f
    t = tensor.reshape(rows, e)
    out_sc = _run_sc(t[:_SC_ROWS].reshape(-1), _SC_ROWS, e, k)
    out_tc = _run_tc(t[_SC_ROWS:], e, k)
    out = jnp.concatenate([out_sc.reshape(_SC_ROWS, e), out_tc], axis=0)
    return out.reshape(b, f, e)
